# baseline (speedup 1.0000x reference)
import functools

import jax
import jax.numpy as jnp
from jax import lax
from jax.experimental import pallas as pl
from jax.experimental.pallas import tpu as pltpu
from jax.experimental.pallas import tpu_sc as plsc

_D = 64
_B = 16384
_NC = 2
_NS = 16
_NW = _NC * _NS
_BPW = _B // _NW
_RPW = 2 * _BPW
_CHUNK = 128
_NCHUNK = _RPW // _CHUNK
_NBUF = 6
_AHEAD = 5

_mesh = plsc.VectorSubcoreMesh(core_axis_name="c", subcore_axis_name="s")


@functools.partial(
    pl.kernel,
    mesh=_mesh,
    out_type=jax.ShapeDtypeStruct((_B * 16,), jnp.float32),
    scratch_types=[
        pltpu.VMEM((_NCHUNK, _CHUNK), jnp.int32),
        pltpu.VMEM((_NBUF, _CHUNK, 1, _D), jnp.float32),
        pltpu.VMEM((_BPW * 16,), jnp.float32),
        pltpu.SemaphoreType.DMA,
    ],
)
def _sc_pair_lorentz(idx_hbm, w3_hbm, out_hbm, idx_v, buf_v, t_v, sem):
    wid = lax.axis_index("s") * _NC + lax.axis_index("c")
    pltpu.sync_copy(idx_hbm.at[wid], idx_v)

    def fire(c):
        return pltpu.async_copy(
            w3_hbm.at[idx_v.at[c]], buf_v.at[c % _NBUF], sem
        )

    copies = {c: fire(c) for c in range(min(_AHEAD, _NCHUNK))}

    lane = lax.iota(jnp.int32, 16)
    sgn = jnp.where(lane == 0, jnp.float32(1.0), jnp.float32(-1.0))

    for c in range(_NCHUNK):
        if c + _AHEAD < _NCHUNK:
            copies[c + _AHEAD] = fire(c + _AHEAD)
        copies[c].wait()
        bu = buf_v.at[c % _NBUF]

        def grp(g, carry, c=c, bu=bu):
            for kk in range(16):
                k = g * 16 + kk
                t = (
                    bu[2 * k, 0, pl.ds(0, 16)]
                    * bu[2 * k + 1, 0, pl.ds(0, 16)]
                    * sgn
                )
                for q in range(1, 4):
                    t = t - (
                        bu[2 * k, 0, pl.ds(q * 16, 16)]
                        * bu[2 * k + 1, 0, pl.ds(q * 16, 16)]
                    )
                t_v[pl.ds(c * 1024 + g * 256 + kk * 16, 16)] = t
            return carry

        lax.fori_loop(0, 4, grp, 0)

    pltpu.sync_copy(t_v, out_hbm.at[pl.ds(wid * _BPW * 16, _BPW * 16)])


def _fold_acosh_body(x_ref, s_ref, o_ref):
    x = x_ref[...]
    col = jax.lax.broadcasted_iota(jnp.int32, (128, 8), 0)
    grp = jax.lax.broadcasted_iota(jnp.int32, (128, 8), 1)
    m = jnp.where(col // 16 == grp, jnp.float32(1.0), jnp.float32(0.0))
    arg = jax.lax.dot_general(
        x, m, (((1,), (0,)), ((), ())),
        precision=jax.lax.Precision.HIGHEST,
        preferred_element_type=jnp.float32,
    )
    arg = jnp.maximum(arg, jnp.float32(1.0 + 1e-9))
    scale = jnp.exp(s_ref[0])
    o_ref[...] = jnp.log(arg + jnp.sqrt((arg - 1.0) * (arg + 1.0))) * scale


@jax.jit
def kernel(idx, w, scale_log):
    idx32 = idx.astype(jnp.int32).reshape(_NW, _NCHUNK, _CHUNK)
    w3 = w[:, None, :]
    partials = _sc_pair_lorentz(idx32, w3)
    out = pl.pallas_call(
        _fold_acosh_body,
        out_shape=jax.ShapeDtypeStruct((_B // 8, 8), jnp.float32),
        in_specs=[
            pl.BlockSpec(memory_space=pltpu.VMEM),
            pl.BlockSpec(memory_space=pltpu.SMEM),
        ],
        out_specs=pl.BlockSpec(memory_space=pltpu.VMEM),
    )(partials.reshape(_B // 8, 128), scale_log)
    return out.reshape(_B)
